# baseline (speedup 1.0000x reference)
import functools

import jax
import jax.numpy as jnp
from jax import lax
from jax.experimental import pallas as pl
from jax.experimental.pallas import tpu as pltpu
from jax.experimental.pallas import tpu_sc as plsc

N_NODES = 10000
N_EDGES = 320000
D_FEAT = 132
D_PAD = 144
D_HALF = 72
WIDTH = 128

NUM_CORES = 2
NUM_SUBCORES = 16

CHUNK = 64
K_CHUNKS = 320
NBUF = 8
E_PAD = NUM_SUBCORES * K_CHUNKS * CHUNK
ROWS_PER_TILE = 640
N_ACC = NUM_SUBCORES * ROWS_PER_TILE
DUMMY_ROW = N_NODES


def _make_agg_kernel():
  mesh = plsc.VectorSubcoreMesh(
      core_axis_name="c", subcore_axis_name="s",
      num_cores=NUM_CORES, num_subcores=NUM_SUBCORES)

  @functools.partial(
      pl.kernel,
      out_type=jax.ShapeDtypeStruct((NUM_CORES, N_ACC, D_HALF), jnp.float32),
      mesh=mesh,
      scratch_types=[
          pltpu.VMEM((K_CHUNKS, CHUNK), jnp.int32),
          pltpu.VMEM((K_CHUNKS, CHUNK), jnp.int32),
          [pltpu.VMEM((CHUNK, D_HALF), jnp.float32) for _ in range(NBUF)],
          pltpu.VMEM_SHARED((N_ACC, D_HALF), jnp.float32),
          [pltpu.SemaphoreType.DMA for _ in range(NBUF)],
          [pltpu.SemaphoreType.DMA for _ in range(NBUF)],
      ],
      compiler_params=pltpu.CompilerParams(use_tc_tiling_on_sc=False),
  )
  def agg(feats_hbm, src_hbm, dst_hbm, zeros_hbm, out_hbm,
          src_v, dst_v, bufs, accum, semg, sems):
    c = lax.axis_index("c")
    s = lax.axis_index("s")
    table = feats_hbm.at[c]

    pltpu.sync_copy(src_hbm.at[s], src_v)
    pltpu.sync_copy(dst_hbm.at[s], dst_v)

    pltpu.sync_copy(zeros_hbm, bufs[0])
    for k in range(ROWS_PER_TILE // CHUNK):
      pltpu.sync_copy(
          bufs[0], accum.at[pl.ds(s * ROWS_PER_TILE + k * CHUNK, CHUNK)])
    plsc.subcore_barrier()

    def gather(j, b):
      return pltpu.make_async_copy(table.at[dst_v.at[j]], bufs[b], semg[b])

    def scatter(j, b):
      return pltpu.make_async_copy(bufs[b], accum.at[src_v.at[j]], sems[b])

    for b in range(NBUF):
      pltpu.async_copy(table.at[dst_v.at[b]], bufs[b], semg[b])

    def body(g, carry):
      j0 = g * NBUF
      for b in range(NBUF):
        gather(j0 + b, b).wait()
        pltpu.async_copy(bufs[b], accum.at[src_v.at[j0 + b]], sems[b],
                         add=True)
      for b in range(NBUF):
        scatter(j0 + b, b).wait()
        pltpu.async_copy(table.at[dst_v.at[j0 + NBUF + b]], bufs[b], semg[b])
      return carry

    lax.fori_loop(0, K_CHUNKS // NBUF - 1, body, 0)
    j0 = K_CHUNKS - NBUF
    for b in range(NBUF):
      gather(j0 + b, b).wait()
      pltpu.async_copy(bufs[b], accum.at[src_v.at[j0 + b]], sems[b], add=True)
    for b in range(NBUF):
      scatter(j0 + b, b).wait()
    plsc.subcore_barrier()

    n_dump = ROWS_PER_TILE // CHUNK
    def rd(k, b):
      base = s * ROWS_PER_TILE + k * CHUNK
      return pltpu.make_async_copy(accum.at[pl.ds(base, CHUNK)], bufs[b],
                                   semg[b])
    def wr(k, b):
      base = s * ROWS_PER_TILE + k * CHUNK
      return pltpu.make_async_copy(bufs[b], out_hbm.at[c].at[pl.ds(base, CHUNK)],
                                   sems[b])
    for k in range(min(NBUF, n_dump)):
      rd(k, k).start()
    for k in range(n_dump):
      b = k % NBUF
      rd(k, b).wait()
      wr(k, b).start()
      if k + NBUF < n_dump:
        wr(k, b).wait()
        rd(k + NBUF, b).start()
    for k in range(max(0, n_dump - NBUF), n_dump):
      wr(k, k % NBUF).wait()

  return agg


def _mlp_body(p_ref, w0a_ref, w0b_ref, b0_ref, w1_ref, b1_ref, w2_ref, b2_ref,
              o_ref):
  sa = p_ref[0, :N_NODES]
  sb = p_ref[1, :N_NODES]
  nrm2 = (jnp.sum(sa * sa, axis=1, keepdims=True)
          + jnp.sum(sb * sb, axis=1, keepdims=True))
  inv = lax.rsqrt(nrm2)
  h = (jnp.dot(sa, w0a_ref[...], preferred_element_type=jnp.float32)
       + jnp.dot(sb, w0b_ref[...], preferred_element_type=jnp.float32))
  h = jnp.maximum(h * inv + b0_ref[...], 0.0)
  h = jnp.maximum(jnp.dot(h, w1_ref[...],
                          preferred_element_type=jnp.float32) + b1_ref[...], 0.0)
  r = jnp.dot(h, w2_ref[...], preferred_element_type=jnp.float32)
  o_ref[...] = (jnp.sum(r) / N_NODES + b2_ref[0, 0]).reshape(1, 1)


def _mlp(partials, W0a, W0b, b0, W1, b1, W2, b2):
  return pl.pallas_call(
      _mlp_body,
      out_shape=jax.ShapeDtypeStruct((1, 1), jnp.float32),
  )(partials, W0a, W0b, b0, W1, b1, W2, b2)


@jax.jit
def kernel(x, pos, z, edge_index, W0, b0, W1, b1, W2, b2):
  pad12 = jnp.zeros((N_NODES, D_HALF - 60), jnp.float32)
  feats = jnp.stack(
      [x[:, :D_HALF],
       jnp.concatenate([x[:, D_HALF:], pos, z[:, None], pad12], axis=1)])

  pad = E_PAD - N_EDGES
  pad_src = DUMMY_ROW + jnp.arange(pad, dtype=jnp.int32) % (N_ACC - DUMMY_ROW)
  pad_blk = jnp.stack([pad_src, jnp.zeros((pad,), jnp.int32)])
  edges = jnp.concatenate([edge_index.astype(jnp.int32), pad_blk], axis=1)
  edges = edges.reshape(2, K_CHUNKS, NUM_SUBCORES, CHUNK).swapaxes(1, 2)
  src = edges[0]
  dst = edges[1]
  zeros_blk = jnp.zeros((CHUNK, D_HALF), jnp.float32)

  partials = _make_agg_kernel()(feats, src, dst, zeros_blk)

  W0a = W0[:D_HALF]
  W0b = jnp.zeros((D_HALF, WIDTH), jnp.float32)
  W0b = W0b.at[:128 - D_HALF].set(W0[D_HALF:128])
  W0b = W0b.at[56:59].set(W0[128:131])
  W0b = W0b.at[59].set(W0[131])
  res = _mlp(partials, W0a, W0b, b0.reshape(1, WIDTH), W1, b1.reshape(1, WIDTH),
             W2, b2.reshape(1, 1))
  return res.reshape(1)


# speedup vs baseline: 1.0051x; 1.0051x over previous
import functools

import jax
import jax.numpy as jnp
from jax import lax
from jax.experimental import pallas as pl
from jax.experimental.pallas import tpu as pltpu
from jax.experimental.pallas import tpu_sc as plsc

N_NODES = 10000
N_EDGES = 320000
D_FEAT = 132
D_PAD = 144
D_HALF = 72
WIDTH = 128

NUM_CORES = 2
NUM_SUBCORES = 16

CHUNK = 128
K_CHUNKS = 160
NBUF = 4
E_PAD = NUM_SUBCORES * K_CHUNKS * CHUNK
ROWS_PER_TILE = 640
N_ACC = NUM_SUBCORES * ROWS_PER_TILE
DUMMY_ROW = N_NODES


def _make_agg_kernel():
  mesh = plsc.VectorSubcoreMesh(
      core_axis_name="c", subcore_axis_name="s",
      num_cores=NUM_CORES, num_subcores=NUM_SUBCORES)

  @functools.partial(
      pl.kernel,
      out_type=jax.ShapeDtypeStruct((NUM_CORES, N_ACC, D_HALF), jnp.float32),
      mesh=mesh,
      scratch_types=[
          pltpu.VMEM((K_CHUNKS, CHUNK), jnp.int32),
          pltpu.VMEM((K_CHUNKS, CHUNK), jnp.int32),
          [pltpu.VMEM((CHUNK, D_HALF), jnp.float32) for _ in range(NBUF)],
          pltpu.VMEM_SHARED((N_ACC, D_HALF), jnp.float32),
          [pltpu.SemaphoreType.DMA for _ in range(NBUF)],
          [pltpu.SemaphoreType.DMA for _ in range(NBUF)],
      ],
      compiler_params=pltpu.CompilerParams(use_tc_tiling_on_sc=False),
  )
  def agg(feats_hbm, edges_hbm, zeros_hbm, out_hbm,
          src_v, dst_v, bufs, accum, semg, sems):
    c = lax.axis_index("c")
    s = lax.axis_index("s")
    table = feats_hbm.at[c]

    pltpu.sync_copy(edges_hbm.at[0, :, s], src_v)
    pltpu.sync_copy(edges_hbm.at[1, :, s], dst_v)

    pltpu.sync_copy(zeros_hbm, bufs[0])
    for k in range(ROWS_PER_TILE // CHUNK):
      pltpu.sync_copy(
          bufs[0], accum.at[pl.ds(s * ROWS_PER_TILE + k * CHUNK, CHUNK)])
    plsc.subcore_barrier()

    def gather(j, b):
      return pltpu.make_async_copy(table.at[dst_v.at[j]], bufs[b], semg[b])

    def scatter(j, b):
      return pltpu.make_async_copy(bufs[b], accum.at[src_v.at[j]], sems[b])

    for b in range(NBUF):
      pltpu.async_copy(table.at[dst_v.at[b]], bufs[b], semg[b])

    def body(g, carry):
      j0 = g * NBUF
      for b in range(NBUF):
        gather(j0 + b, b).wait()
        pltpu.async_copy(bufs[b], accum.at[src_v.at[j0 + b]], sems[b],
                         add=True)
      for b in range(NBUF):
        scatter(j0 + b, b).wait()
        pltpu.async_copy(table.at[dst_v.at[j0 + NBUF + b]], bufs[b], semg[b])
      return carry

    lax.fori_loop(0, K_CHUNKS // NBUF - 1, body, 0)
    j0 = K_CHUNKS - NBUF
    for b in range(NBUF):
      gather(j0 + b, b).wait()
      pltpu.async_copy(bufs[b], accum.at[src_v.at[j0 + b]], sems[b], add=True)
    for b in range(NBUF):
      scatter(j0 + b, b).wait()
    plsc.subcore_barrier()

    n_dump = ROWS_PER_TILE // CHUNK
    def rd(k, b):
      base = s * ROWS_PER_TILE + k * CHUNK
      return pltpu.make_async_copy(accum.at[pl.ds(base, CHUNK)], bufs[b],
                                   semg[b])
    def wr(k, b):
      base = s * ROWS_PER_TILE + k * CHUNK
      return pltpu.make_async_copy(bufs[b], out_hbm.at[c].at[pl.ds(base, CHUNK)],
                                   sems[b])
    for k in range(min(NBUF, n_dump)):
      rd(k, k).start()
    for k in range(n_dump):
      b = k % NBUF
      rd(k, b).wait()
      wr(k, b).start()
      if k + NBUF < n_dump:
        wr(k, b).wait()
        rd(k + NBUF, b).start()
    for k in range(max(0, n_dump - NBUF), n_dump):
      wr(k, k % NBUF).wait()

  return agg


def _mlp_body(p_hbm, w0a_ref, w0b_ref, b0_ref, w1_ref, b1_ref, w2_ref, b2_ref,
              o_ref, p_ref, sem):
  pltpu.async_copy(p_hbm, p_ref, sem).wait()
  sa = p_ref[0, :N_NODES]
  sb = p_ref[1, :N_NODES]
  nrm2 = (jnp.sum(sa * sa, axis=1, keepdims=True)
          + jnp.sum(sb * sb, axis=1, keepdims=True))
  inv = lax.rsqrt(nrm2)
  h = (jnp.dot(sa, w0a_ref[...], preferred_element_type=jnp.float32)
       + jnp.dot(sb, w0b_ref[...], preferred_element_type=jnp.float32))
  h = jnp.maximum(h * inv + b0_ref[...], 0.0)
  h = jnp.maximum(jnp.dot(h, w1_ref[...],
                          preferred_element_type=jnp.float32) + b1_ref[...], 0.0)
  r = jnp.dot(h, w2_ref[...], preferred_element_type=jnp.float32)
  o_ref[...] = (jnp.sum(r) / N_NODES + b2_ref[0, 0]).reshape(1, 1)


def _mlp(partials, W0a, W0b, b0, W1, b1, W2, b2):
  return pl.pallas_call(
      _mlp_body,
      out_shape=jax.ShapeDtypeStruct((1, 1), jnp.float32),
      in_specs=[pl.BlockSpec(memory_space=pl.ANY)] + [
          pl.BlockSpec(memory_space=pltpu.VMEM) for _ in range(7)],
      scratch_shapes=[
          pltpu.VMEM((NUM_CORES, N_ACC, D_HALF), jnp.float32),
          pltpu.SemaphoreType.DMA,
      ],
  )(partials, W0a, W0b, b0, W1, b1, W2, b2)


@jax.jit
def kernel(x, pos, z, edge_index, W0, b0, W1, b1, W2, b2):
  pad12 = jnp.zeros((N_NODES, D_HALF - 60), jnp.float32)
  feats = jnp.stack(
      [x[:, :D_HALF],
       jnp.concatenate([x[:, D_HALF:], pos, z[:, None], pad12], axis=1)])

  pad = E_PAD - N_EDGES
  pad_src = DUMMY_ROW + jnp.arange(pad, dtype=jnp.int32) % (N_ACC - DUMMY_ROW)
  pad_blk = jnp.stack([pad_src, jnp.zeros((pad,), jnp.int32)])
  edges = jnp.concatenate([edge_index.astype(jnp.int32), pad_blk], axis=1)
  edges = edges.reshape(2, K_CHUNKS, NUM_SUBCORES, CHUNK)
  zeros_blk = jnp.zeros((CHUNK, D_HALF), jnp.float32)

  partials = _make_agg_kernel()(feats, edges, zeros_blk)

  W0a = W0[:D_HALF]
  W0b = jnp.zeros((D_HALF, WIDTH), jnp.float32)
  W0b = W0b.at[:128 - D_HALF].set(W0[D_HALF:128])
  W0b = W0b.at[56:59].set(W0[128:131])
  W0b = W0b.at[59].set(W0[131])
  res = _mlp(partials, W0a, W0b, b0.reshape(1, WIDTH), W1, b1.reshape(1, WIDTH),
             W2, b2.reshape(1, 1))
  return res.reshape(1)


# speedup vs baseline: 1.5394x; 1.5316x over previous
import functools

import jax
import jax.numpy as jnp
from jax import lax
from jax.experimental import pallas as pl
from jax.experimental.pallas import tpu as pltpu
from jax.experimental.pallas import tpu_sc as plsc

N_NODES = 10000
N_EDGES = 320000
D_FEAT = 132
D_PAD = 144
D_HALF = 72
WIDTH = 128

NUM_CORES = 2
NUM_SUBCORES = 16

CHUNK = 128
K_CHUNKS = 160
NBUF = 2
ROWS_PER_LOAD = N_NODES // NUM_SUBCORES
E_PAD = NUM_SUBCORES * K_CHUNKS * CHUNK
ROWS_PER_TILE = 640
N_ACC = NUM_SUBCORES * ROWS_PER_TILE
DUMMY_ROW = N_NODES


def _make_agg_kernel():
  mesh = plsc.VectorSubcoreMesh(
      core_axis_name="c", subcore_axis_name="s",
      num_cores=NUM_CORES, num_subcores=NUM_SUBCORES)

  @functools.partial(
      pl.kernel,
      out_type=jax.ShapeDtypeStruct((NUM_CORES, N_ACC, D_HALF), jnp.float32),
      mesh=mesh,
      scratch_types=[
          pltpu.VMEM((K_CHUNKS, CHUNK), jnp.int32),
          pltpu.VMEM((NBUF, CHUNK), jnp.int32),
          pltpu.VMEM((NBUF, CHUNK), jnp.int32),
          [pltpu.VMEM((CHUNK, D_HALF), jnp.float32) for _ in range(NBUF)],
          pltpu.VMEM_SHARED((N_NODES, D_HALF), jnp.float32),
          pltpu.VMEM_SHARED((N_ACC, D_HALF), jnp.float32),
          [pltpu.SemaphoreType.DMA for _ in range(NBUF)],
          [pltpu.SemaphoreType.DMA for _ in range(NBUF)],
      ],
      compiler_params=pltpu.CompilerParams(use_tc_tiling_on_sc=False),
  )
  def agg(feats_hbm, packed_hbm, zeros_hbm, out_hbm,
          packed_v, src_u, dst_u, bufs, table, accum, semg, sems):
    c = lax.axis_index("c")
    s = lax.axis_index("s")

    pltpu.sync_copy(packed_hbm.at[:, s], packed_v)

    pltpu.sync_copy(feats_hbm.at[c].at[pl.ds(s * ROWS_PER_LOAD, ROWS_PER_LOAD)],
                    table.at[pl.ds(s * ROWS_PER_LOAD, ROWS_PER_LOAD)])

    pltpu.sync_copy(zeros_hbm, bufs[0])
    for k in range(ROWS_PER_TILE // CHUNK):
      pltpu.sync_copy(
          bufs[0], accum.at[pl.ds(s * ROWS_PER_TILE + k * CHUNK, CHUNK)])
    plsc.subcore_barrier()

    def unpack(j, b):
      for k in range(CHUNK // 16):
        v = packed_v[j, pl.ds(16 * k, 16)]
        src_u[b, pl.ds(16 * k, 16)] = jax.lax.shift_right_logical(v, 16)
        dst_u[b, pl.ds(16 * k, 16)] = jax.lax.bitwise_and(v, 0xFFFF)

    def gather(j, b):
      return pltpu.make_async_copy(table.at[dst_u.at[b]], bufs[b], semg[b])

    def scatter(j, b):
      return pltpu.make_async_copy(bufs[b], accum.at[src_u.at[b]], sems[b])

    for b in range(NBUF):
      unpack(b, b)
      pltpu.async_copy(table.at[dst_u.at[b]], bufs[b], semg[b])

    def body(g, carry):
      j0 = g * NBUF
      for b in range(NBUF):
        gather(j0 + b, b).wait()
        pltpu.async_copy(bufs[b], accum.at[src_u.at[b]], sems[b], add=True)
      for b in range(NBUF):
        scatter(j0 + b, b).wait()
        unpack(j0 + NBUF + b, b)
        pltpu.async_copy(table.at[dst_u.at[b]], bufs[b], semg[b])
      return carry

    lax.fori_loop(0, K_CHUNKS // NBUF - 1, body, 0)
    j0 = K_CHUNKS - NBUF
    for b in range(NBUF):
      gather(j0 + b, b).wait()
      pltpu.async_copy(bufs[b], accum.at[src_u.at[b]], sems[b], add=True)
    for b in range(NBUF):
      scatter(j0 + b, b).wait()
    plsc.subcore_barrier()

    n_dump = ROWS_PER_TILE // CHUNK
    def rd(k, b):
      base = s * ROWS_PER_TILE + k * CHUNK
      return pltpu.make_async_copy(accum.at[pl.ds(base, CHUNK)], bufs[b],
                                   semg[b])
    def wr(k, b):
      base = s * ROWS_PER_TILE + k * CHUNK
      return pltpu.make_async_copy(bufs[b], out_hbm.at[c].at[pl.ds(base, CHUNK)],
                                   sems[b])
    for k in range(min(NBUF, n_dump)):
      rd(k, k).start()
    for k in range(n_dump):
      b = k % NBUF
      rd(k, b).wait()
      wr(k, b).start()
      if k + NBUF < n_dump:
        wr(k, b).wait()
        rd(k + NBUF, b).start()
    for k in range(max(0, n_dump - NBUF), n_dump):
      wr(k, k % NBUF).wait()

  return agg


def _mlp_body(p_hbm, w0a_ref, w0b_ref, b0_ref, w1_ref, b1_ref, w2_ref, b2_ref,
              o_ref, p_ref, sem):
  pltpu.async_copy(p_hbm, p_ref, sem).wait()
  sa = p_ref[0, :N_NODES]
  sb = p_ref[1, :N_NODES]
  nrm2 = (jnp.sum(sa * sa, axis=1, keepdims=True)
          + jnp.sum(sb * sb, axis=1, keepdims=True))
  inv = lax.rsqrt(nrm2)
  h = (jnp.dot(sa, w0a_ref[...], preferred_element_type=jnp.float32)
       + jnp.dot(sb, w0b_ref[...], preferred_element_type=jnp.float32))
  h = jnp.maximum(h * inv + b0_ref[...], 0.0)
  h = jnp.maximum(jnp.dot(h, w1_ref[...],
                          preferred_element_type=jnp.float32) + b1_ref[...], 0.0)
  r = jnp.dot(h, w2_ref[...], preferred_element_type=jnp.float32)
  o_ref[...] = (jnp.sum(r) / N_NODES + b2_ref[0, 0]).reshape(1, 1)


def _mlp(partials, W0a, W0b, b0, W1, b1, W2, b2):
  return pl.pallas_call(
      _mlp_body,
      out_shape=jax.ShapeDtypeStruct((1, 1), jnp.float32),
      in_specs=[pl.BlockSpec(memory_space=pl.ANY)] + [
          pl.BlockSpec(memory_space=pltpu.VMEM) for _ in range(7)],
      scratch_shapes=[
          pltpu.VMEM((NUM_CORES, N_ACC, D_HALF), jnp.float32),
          pltpu.SemaphoreType.DMA,
      ],
  )(partials, W0a, W0b, b0, W1, b1, W2, b2)


@jax.jit
def kernel(x, pos, z, edge_index, W0, b0, W1, b1, W2, b2):
  pad12 = jnp.zeros((N_NODES, D_HALF - 60), jnp.float32)
  feats = jnp.stack(
      [x[:, :D_HALF],
       jnp.concatenate([x[:, D_HALF:], pos, z[:, None], pad12], axis=1)])

  pad = E_PAD - N_EDGES
  pad_src = DUMMY_ROW + jnp.arange(pad, dtype=jnp.int32) % (N_ACC - DUMMY_ROW)
  pad_blk = jnp.stack([pad_src, jnp.zeros((pad,), jnp.int32)])
  edges = jnp.concatenate([edge_index.astype(jnp.int32), pad_blk], axis=1)
  packed = jnp.left_shift(edges[0], 16) | edges[1]
  packed = packed.reshape(K_CHUNKS, NUM_SUBCORES, CHUNK)
  zeros_blk = jnp.zeros((CHUNK, D_HALF), jnp.float32)

  partials = _make_agg_kernel()(feats, packed, zeros_blk)

  W0a = W0[:D_HALF]
  W0b = jnp.zeros((D_HALF, WIDTH), jnp.float32)
  W0b = W0b.at[:128 - D_HALF].set(W0[D_HALF:128])
  W0b = W0b.at[56:59].set(W0[128:131])
  W0b = W0b.at[59].set(W0[131])
  res = _mlp(partials, W0a, W0b, b0.reshape(1, WIDTH), W1, b1.reshape(1, WIDTH),
             W2, b2.reshape(1, 1))
  return res.reshape(1)


# speedup vs baseline: 1.6693x; 1.0844x over previous
import functools

import jax
import jax.numpy as jnp
from jax import lax
from jax.experimental import pallas as pl
from jax.experimental.pallas import tpu as pltpu
from jax.experimental.pallas import tpu_sc as plsc

N_NODES = 10000
N_EDGES = 320000
D_FEAT = 132
D_PAD = 144
D_HALF = 72
WIDTH = 128

NUM_CORES = 2
NUM_SUBCORES = 16

CHUNK = 128
K_CHUNKS = 160
NBUF = 2
ROWS_PER_LOAD = N_NODES // NUM_SUBCORES
E_PAD = NUM_SUBCORES * K_CHUNKS * CHUNK
ROWS_PER_TILE = 640
N_ACC = NUM_SUBCORES * ROWS_PER_TILE
DUMMY_ROW = N_NODES


def _make_agg_kernel():
  mesh = plsc.VectorSubcoreMesh(
      core_axis_name="c", subcore_axis_name="s",
      num_cores=NUM_CORES, num_subcores=NUM_SUBCORES)

  @functools.partial(
      pl.kernel,
      out_type=jax.ShapeDtypeStruct((NUM_CORES, N_ACC, D_HALF), jnp.float32),
      mesh=mesh,
      scratch_types=[
          pltpu.VMEM((K_CHUNKS, CHUNK), jnp.int32),
          pltpu.VMEM((NBUF, CHUNK), jnp.int32),
          pltpu.VMEM((NBUF, CHUNK), jnp.int32),
          [pltpu.VMEM((CHUNK, D_HALF), jnp.float32) for _ in range(NBUF)],
          pltpu.VMEM_SHARED((N_NODES, D_HALF), jnp.float32),
          pltpu.VMEM_SHARED((N_ACC, D_HALF), jnp.float32),
          [pltpu.SemaphoreType.DMA for _ in range(NBUF)],
          [pltpu.SemaphoreType.DMA for _ in range(NBUF)],
      ],
      compiler_params=pltpu.CompilerParams(use_tc_tiling_on_sc=False),
  )
  def agg(x_hbm, posz_hbm, packed_hbm, zeros_hbm, out_hbm,
          packed_v, src_u, dst_u, bufs, table, accum, semg, sems):
    c = lax.axis_index("c")
    s = lax.axis_index("s")

    pltpu.sync_copy(packed_hbm.at[:, s], packed_v)

    rows = pl.ds(s * ROWS_PER_LOAD, ROWS_PER_LOAD)

    @pl.when(c == 0)
    def _():
      pltpu.sync_copy(x_hbm.at[rows, pl.ds(0, D_HALF)], table.at[rows])

    @pl.when(c == 1)
    def _():
      pltpu.sync_copy(x_hbm.at[rows, pl.ds(D_HALF, 128 - D_HALF)],
                      table.at[rows, pl.ds(0, 128 - D_HALF)])
      pltpu.sync_copy(posz_hbm.at[rows],
                      table.at[rows, pl.ds(128 - D_HALF, 8)])
      for k in range(ROWS_PER_LOAD // 125):
        pltpu.sync_copy(
            zeros_hbm.at[pl.ds(0, 125), pl.ds(0, 8)],
            table.at[pl.ds(s * ROWS_PER_LOAD + k * 125, 125), pl.ds(64, 8)])

    pltpu.sync_copy(zeros_hbm, bufs[0])
    for k in range(ROWS_PER_TILE // CHUNK):
      pltpu.sync_copy(
          bufs[0], accum.at[pl.ds(s * ROWS_PER_TILE + k * CHUNK, CHUNK)])
    plsc.subcore_barrier()

    def unpack(j, b):
      for k in range(CHUNK // 16):
        v = packed_v[j, pl.ds(16 * k, 16)]
        src_u[b, pl.ds(16 * k, 16)] = jax.lax.shift_right_logical(v, 16)
        dst_u[b, pl.ds(16 * k, 16)] = jax.lax.bitwise_and(v, 0xFFFF)

    def gather(j, b):
      return pltpu.make_async_copy(table.at[dst_u.at[b]], bufs[b], semg[b])

    def scatter(j, b):
      return pltpu.make_async_copy(bufs[b], accum.at[src_u.at[b]], sems[b])

    for b in range(NBUF):
      unpack(b, b)
      pltpu.async_copy(table.at[dst_u.at[b]], bufs[b], semg[b])

    def body(g, carry):
      j0 = g * NBUF
      for b in range(NBUF):
        gather(j0 + b, b).wait()
        pltpu.async_copy(bufs[b], accum.at[src_u.at[b]], sems[b], add=True)
      for b in range(NBUF):
        scatter(j0 + b, b).wait()
        unpack(j0 + NBUF + b, b)
        pltpu.async_copy(table.at[dst_u.at[b]], bufs[b], semg[b])
      return carry

    lax.fori_loop(0, K_CHUNKS // NBUF - 1, body, 0)
    j0 = K_CHUNKS - NBUF
    for b in range(NBUF):
      gather(j0 + b, b).wait()
      pltpu.async_copy(bufs[b], accum.at[src_u.at[b]], sems[b], add=True)
    for b in range(NBUF):
      scatter(j0 + b, b).wait()
    plsc.subcore_barrier()

    n_dump = ROWS_PER_TILE // CHUNK
    def rd(k, b):
      base = s * ROWS_PER_TILE + k * CHUNK
      return pltpu.make_async_copy(accum.at[pl.ds(base, CHUNK)], bufs[b],
                                   semg[b])
    def wr(k, b):
      base = s * ROWS_PER_TILE + k * CHUNK
      return pltpu.make_async_copy(bufs[b], out_hbm.at[c].at[pl.ds(base, CHUNK)],
                                   sems[b])
    for k in range(min(NBUF, n_dump)):
      rd(k, k).start()
    for k in range(n_dump):
      b = k % NBUF
      rd(k, b).wait()
      wr(k, b).start()
      if k + NBUF < n_dump:
        wr(k, b).wait()
        rd(k + NBUF, b).start()
    for k in range(max(0, n_dump - NBUF), n_dump):
      wr(k, k % NBUF).wait()

  return agg


def _mlp_body(p_hbm, w0a_ref, w0b_ref, b0_ref, w1_ref, b1_ref, w2_ref, b2_ref,
              o_ref, p_ref, sem):
  pltpu.async_copy(p_hbm, p_ref, sem).wait()
  sa = p_ref[0, :N_NODES]
  sb = p_ref[1, :N_NODES]
  sbr = sb[:, :60]
  nrm2 = (jnp.sum(sa * sa, axis=1, keepdims=True)
          + jnp.sum(sbr * sbr, axis=1, keepdims=True))
  inv = lax.rsqrt(nrm2)
  h = (jnp.dot(sa, w0a_ref[...], preferred_element_type=jnp.float32)
       + jnp.dot(sb, w0b_ref[...], preferred_element_type=jnp.float32))
  h = jnp.maximum(h * inv + b0_ref[...], 0.0)
  h = jnp.maximum(jnp.dot(h, w1_ref[...],
                          preferred_element_type=jnp.float32) + b1_ref[...], 0.0)
  r = jnp.dot(h, w2_ref[...], preferred_element_type=jnp.float32)
  o_ref[...] = (jnp.sum(r) / N_NODES + b2_ref[0, 0]).reshape(1, 1)


def _mlp(partials, W0a, W0b, b0, W1, b1, W2, b2):
  return pl.pallas_call(
      _mlp_body,
      out_shape=jax.ShapeDtypeStruct((1, 1), jnp.float32),
      in_specs=[pl.BlockSpec(memory_space=pl.ANY)] + [
          pl.BlockSpec(memory_space=pltpu.VMEM) for _ in range(7)],
      scratch_shapes=[
          pltpu.VMEM((NUM_CORES, N_ACC, D_HALF), jnp.float32),
          pltpu.SemaphoreType.DMA,
      ],
  )(partials, W0a, W0b, b0, W1, b1, W2, b2)


@jax.jit
def kernel(x, pos, z, edge_index, W0, b0, W1, b1, W2, b2):
  posz = jnp.concatenate(
      [pos, z[:, None], jnp.zeros((N_NODES, 4), jnp.float32)], axis=1)

  pad = E_PAD - N_EDGES
  pad_src = DUMMY_ROW + jnp.arange(pad, dtype=jnp.int32) % (N_ACC - DUMMY_ROW)
  pad_blk = jnp.stack([pad_src, jnp.zeros((pad,), jnp.int32)])
  edges = jnp.concatenate([edge_index.astype(jnp.int32), pad_blk], axis=1)
  packed = jnp.left_shift(edges[0], 16) | edges[1]
  packed = packed.reshape(K_CHUNKS, NUM_SUBCORES, CHUNK)
  zeros_blk = jnp.zeros((CHUNK, D_HALF), jnp.float32)

  partials = _make_agg_kernel()(x, posz, packed, zeros_blk)

  W0a = W0[:D_HALF]
  W0b = jnp.zeros((D_HALF, WIDTH), jnp.float32)
  W0b = W0b.at[:128 - D_HALF].set(W0[D_HALF:128])
  W0b = W0b.at[56:59].set(W0[128:131])
  W0b = W0b.at[59].set(W0[131])
  res = _mlp(partials, W0a, W0b, b0.reshape(1, WIDTH), W1, b1.reshape(1, WIDTH),
             W2, b2.reshape(1, 1))
  return res.reshape(1)
